# baseline (speedup 1.0000x reference)
import functools

import jax
import jax.numpy as jnp
from jax import lax
from jax.experimental import pallas as pl
from jax.experimental.pallas import tpu as pltpu
from jax.experimental.pallas import tpu_sc as plsc

CH = 128
NC = 2
NS = 16
NW = NC * NS
CHUNK = 128
STAGE = 16

_mesh = plsc.VectorSubcoreMesh(
    core_axis_name="c", subcore_axis_name="s", num_cores=NC, num_subcores=NS)


def _pad_node_count(n):
    m = 16 * NS
    return ((n + 1 + m - 1) // m) * m


def _make_deg_kernel(n_pad, ep_w):
    n_pt = n_pad // NS

    @functools.partial(
        pl.kernel,
        out_type=jax.ShapeDtypeStruct((NC, n_pad), jnp.float32),
        mesh=_mesh,
        scratch_types=[
            pltpu.VMEM((CHUNK,), jnp.int32),
            pltpu.VMEM((CHUNK,), jnp.float32),
            pltpu.VMEM((n_pt,), jnp.float32),
            pltpu.VMEM_SHARED((n_pad,), jnp.float32),
        ],
    )
    def deg_kernel(dst_hbm, out_hbm, didx_v, ones_v, zbuf_v, acc_sh):
        cid = lax.axis_index("c")
        sid = lax.axis_index("s")
        wid = cid * NS + sid
        for i in range(CHUNK // 16):
            ones_v[pl.ds(i * 16, 16)] = jnp.ones((16,), jnp.float32)
        for i in range(n_pt // 16):
            zbuf_v[pl.ds(i * 16, 16)] = jnp.zeros((16,), jnp.float32)
        pltpu.sync_copy(zbuf_v, acc_sh.at[pl.ds(sid * n_pt, n_pt)])
        plsc.subcore_barrier()

        def body(i, carry):
            base = wid * ep_w + i * CHUNK
            pltpu.sync_copy(dst_hbm.at[pl.ds(base, CHUNK)], didx_v)
            pltpu.sync_copy(ones_v, acc_sh.at[didx_v], add=True)
            return carry

        lax.fori_loop(0, ep_w // CHUNK, body, 0)
        plsc.subcore_barrier()
        pltpu.sync_copy(acc_sh.at[pl.ds(sid * n_pt, n_pt)],
                        out_hbm.at[cid, pl.ds(sid * n_pt, n_pt)])

    return deg_kernel


def _make_edge_kernel(n_pad, cpt):
    n_pt = n_pad // NS

    @functools.partial(
        pl.kernel,
        out_type=jax.ShapeDtypeStruct((NC, n_pad, CH), jnp.float32),
        mesh=_mesh,
        scratch_types=[
            pltpu.VMEM((STAGE, CHUNK), jnp.int32),
            pltpu.VMEM((STAGE, CHUNK), jnp.int32),
            pltpu.VMEM((CHUNK, CH), jnp.float32),
            pltpu.VMEM((CHUNK, CH), jnp.float32),
            pltpu.VMEM_SHARED((n_pad, CH), jnp.float32),
            pltpu.SemaphoreType.DMA,
            pltpu.SemaphoreType.DMA,
        ],
    )
    def edge_kernel(tbl_hbm, src_hbm, dst_hbm, zeros_hbm, out_hbm,
                    sidx, didx, rows_a, rows_b, acc_sh, gsem_a, gsem_b):
        cid = lax.axis_index("c")
        sid = lax.axis_index("s")
        wid = cid * NS + sid
        pltpu.sync_copy(zeros_hbm.at[pl.ds(sid * n_pt, n_pt), :],
                        acc_sh.at[pl.ds(sid * n_pt, n_pt), :])
        plsc.subcore_barrier()

        def stage(s, carry):
            base_row = wid * cpt + s * STAGE
            pltpu.sync_copy(src_hbm.at[pl.ds(base_row, STAGE), :], sidx)
            pltpu.sync_copy(dst_hbm.at[pl.ds(base_row, STAGE), :], didx)
            pltpu.async_copy(tbl_hbm.at[sidx.at[0]], rows_a, gsem_a)

            def body(j, c):
                a = 2 * j
                b = a + 1
                pltpu.make_async_copy(tbl_hbm.at[sidx.at[a]], rows_a, gsem_a).wait()
                pltpu.async_copy(tbl_hbm.at[sidx.at[b]], rows_b, gsem_b)
                pltpu.sync_copy(rows_a, acc_sh.at[didx.at[a]], add=True)
                pltpu.make_async_copy(tbl_hbm.at[sidx.at[b]], rows_b, gsem_b).wait()
                nxt = jnp.minimum(a + 2, STAGE - 1)
                pltpu.async_copy(tbl_hbm.at[sidx.at[nxt]], rows_a, gsem_a)
                pltpu.sync_copy(rows_b, acc_sh.at[didx.at[b]], add=True)
                return c

            lax.fori_loop(0, STAGE // 2, body, 0)
            pltpu.make_async_copy(tbl_hbm.at[sidx.at[0]], rows_a, gsem_a).wait()
            return carry

        lax.fori_loop(0, cpt // STAGE, stage, 0)
        plsc.subcore_barrier()
        pltpu.sync_copy(acc_sh.at[pl.ds(sid * n_pt, n_pt), :],
                        out_hbm.at[cid, pl.ds(sid * n_pt, n_pt), :])

    return edge_kernel


def _tc_prescale_body(deg_ref, x_ref, w_ref, dinv_ref, h_ref):
    d = deg_ref[...]
    dinv = lax.rsqrt(d[0] + d[1] + 1.0)
    dinv_ref[...] = dinv
    h = jnp.dot(x_ref[...], w_ref[...], preferred_element_type=jnp.float32)
    h_ref[...] = h * dinv


def _tc_mid_body(acc_ref, hp_ref, dinv_ref, b_ref, w_ref, out_ref):
    a = acc_ref[...]
    dinv = dinv_ref[...]
    agg = (a[0] + a[1] + hp_ref[...]) * dinv + b_ref[...]
    h2 = jnp.maximum(agg, 0.0)
    out_ref[...] = jnp.dot(h2, w_ref[...], preferred_element_type=jnp.float32) * dinv


def _tc_final_body(acc_ref, hp_ref, dinv_ref, b_ref, out_ref):
    a = acc_ref[...]
    out_ref[...] = (a[0] + a[1] + hp_ref[...]) * dinv_ref[...] + b_ref[...]


def kernel(x, edge_index, W1, b1, W2, b2):
    n = x.shape[0]
    e = edge_index.shape[1]
    n_pad = _pad_node_count(n)
    cpt = ((e + NW * CHUNK - 1) // (NW * CHUNK) + STAGE - 1) // STAGE * STAGE
    e_pad = NW * CHUNK * cpt
    ep_w = e_pad // NW

    src = edge_index[0].astype(jnp.int32)
    dst = edge_index[1].astype(jnp.int32)
    pad_idx = jnp.full((e_pad - e,), n, jnp.int32)
    src_p = jnp.concatenate([src, pad_idx])
    dst_p = jnp.concatenate([dst, pad_idx])
    x_p = jnp.pad(x, ((0, n_pad - n), (0, 0)))
    zeros = jnp.zeros((n_pad, CH), jnp.float32)

    src2 = src_p.reshape(e_pad // CHUNK, CHUNK)
    dst2 = dst_p.reshape(e_pad // CHUNK, CHUNK)

    deg_kernel = _make_deg_kernel(n_pad, ep_w)
    edge_kernel = _make_edge_kernel(n_pad, cpt)

    deg_p = deg_kernel(dst_p).reshape(NC, n_pad, 1)

    dinv, h1p = pl.pallas_call(
        _tc_prescale_body,
        out_shape=[
            jax.ShapeDtypeStruct((n_pad, 1), jnp.float32),
            jax.ShapeDtypeStruct((n_pad, CH), jnp.float32),
        ],
    )(deg_p, x_p, W1)

    acc1 = edge_kernel(h1p, src2, dst2, zeros)

    h2p = pl.pallas_call(
        _tc_mid_body,
        out_shape=jax.ShapeDtypeStruct((n_pad, CH), jnp.float32),
    )(acc1, h1p, dinv, b1.reshape(1, CH), W2)

    acc2 = edge_kernel(h2p, src2, dst2, zeros)

    out = pl.pallas_call(
        _tc_final_body,
        out_shape=jax.ShapeDtypeStruct((n_pad, CH), jnp.float32),
    )(acc2, h2p, dinv, b2.reshape(1, CH))

    return out[:n]


# speedup vs baseline: 1.0718x; 1.0718x over previous
import functools

import jax
import jax.numpy as jnp
from jax import lax
from jax.experimental import pallas as pl
from jax.experimental.pallas import tpu as pltpu
from jax.experimental.pallas import tpu_sc as plsc

CH = 128
NC = 2
NS = 16
NW = NC * NS
CHUNK = 128
STAGE = 16

_mesh = plsc.VectorSubcoreMesh(
    core_axis_name="c", subcore_axis_name="s", num_cores=NC, num_subcores=NS)


def _pad_node_count(n):
    m = 16 * NS
    return ((n + 1 + m - 1) // m) * m


def _make_deg_kernel(n_pad, ep_w):
    n_pt = n_pad // NS

    @functools.partial(
        pl.kernel,
        out_type=jax.ShapeDtypeStruct((NC, n_pad), jnp.float32),
        mesh=_mesh,
        scratch_types=[
            pltpu.VMEM((CHUNK,), jnp.int32),
            pltpu.VMEM((CHUNK,), jnp.float32),
            pltpu.VMEM((n_pt,), jnp.float32),
            pltpu.VMEM_SHARED((n_pad,), jnp.float32),
        ],
    )
    def deg_kernel(dst_hbm, out_hbm, didx_v, ones_v, zbuf_v, acc_sh):
        cid = lax.axis_index("c")
        sid = lax.axis_index("s")
        wid = cid * NS + sid
        for i in range(CHUNK // 16):
            ones_v[pl.ds(i * 16, 16)] = jnp.ones((16,), jnp.float32)
        for i in range(n_pt // 16):
            zbuf_v[pl.ds(i * 16, 16)] = jnp.zeros((16,), jnp.float32)
        pltpu.sync_copy(zbuf_v, acc_sh.at[pl.ds(sid * n_pt, n_pt)])
        plsc.subcore_barrier()

        def body(i, carry):
            base = wid * ep_w + i * CHUNK
            pltpu.sync_copy(dst_hbm.at[pl.ds(base, CHUNK)], didx_v)
            pltpu.sync_copy(ones_v, acc_sh.at[didx_v], add=True)
            return carry

        lax.fori_loop(0, ep_w // CHUNK, body, 0)
        plsc.subcore_barrier()
        pltpu.sync_copy(acc_sh.at[pl.ds(sid * n_pt, n_pt)],
                        out_hbm.at[cid, pl.ds(sid * n_pt, n_pt)])

    return deg_kernel


def _make_edge_kernel(n_pad, cpt):
    n_pt = n_pad // NS
    ep_w = cpt * CHUNK

    @functools.partial(
        pl.kernel,
        out_type=jax.ShapeDtypeStruct((NC, n_pad, CH), jnp.float32),
        mesh=_mesh,
        scratch_types=[
            pltpu.VMEM((CHUNK,), jnp.int32),
            pltpu.VMEM((CHUNK,), jnp.int32),
            pltpu.VMEM((CHUNK,), jnp.int32),
            pltpu.VMEM((CHUNK,), jnp.int32),
            pltpu.VMEM((CHUNK, CH), jnp.float32),
            pltpu.VMEM((CHUNK, CH), jnp.float32),
            pltpu.VMEM_SHARED((n_pad, CH), jnp.float32),
            pltpu.SemaphoreType.DMA,
            pltpu.SemaphoreType.DMA,
        ],
    )
    def edge_kernel(tbl_hbm, src_hbm, dst_hbm, zeros_hbm, out_hbm,
                    sidx_a, didx_a, sidx_b, didx_b, rows_a, rows_b,
                    acc_sh, gsem_a, gsem_b):
        cid = lax.axis_index("c")
        sid = lax.axis_index("s")
        wid = cid * NS + sid
        base0 = wid * ep_w
        pltpu.sync_copy(zeros_hbm.at[pl.ds(sid * n_pt, n_pt), :],
                        acc_sh.at[pl.ds(sid * n_pt, n_pt), :])
        plsc.subcore_barrier()

        pltpu.sync_copy(src_hbm.at[pl.ds(base0, CHUNK)], sidx_a)
        pltpu.sync_copy(dst_hbm.at[pl.ds(base0, CHUNK)], didx_a)
        pltpu.async_copy(tbl_hbm.at[sidx_a], rows_a, gsem_a)

        def body(j, c):
            a = 2 * j
            b = a + 1
            pltpu.sync_copy(src_hbm.at[pl.ds(base0 + b * CHUNK, CHUNK)], sidx_b)
            pltpu.sync_copy(dst_hbm.at[pl.ds(base0 + b * CHUNK, CHUNK)], didx_b)
            pltpu.make_async_copy(tbl_hbm.at[sidx_a], rows_a, gsem_a).wait()
            pltpu.async_copy(tbl_hbm.at[sidx_b], rows_b, gsem_b)
            pltpu.sync_copy(rows_a, acc_sh.at[didx_a], add=True)
            nxt = base0 + jnp.minimum(a + 2, cpt - 1) * CHUNK
            pltpu.sync_copy(src_hbm.at[pl.ds(nxt, CHUNK)], sidx_a)
            pltpu.sync_copy(dst_hbm.at[pl.ds(nxt, CHUNK)], didx_a)
            pltpu.make_async_copy(tbl_hbm.at[sidx_b], rows_b, gsem_b).wait()
            pltpu.async_copy(tbl_hbm.at[sidx_a], rows_a, gsem_a)
            pltpu.sync_copy(rows_b, acc_sh.at[didx_b], add=True)
            return c

        lax.fori_loop(0, cpt // 2, body, 0)
        pltpu.make_async_copy(tbl_hbm.at[sidx_a], rows_a, gsem_a).wait()
        plsc.subcore_barrier()
        pltpu.sync_copy(acc_sh.at[pl.ds(sid * n_pt, n_pt), :],
                        out_hbm.at[cid, pl.ds(sid * n_pt, n_pt), :])

    return edge_kernel


def _tc_prescale_body(deg_ref, x_ref, w_ref, dinv_ref, h_ref):
    d = deg_ref[...]
    dinv = lax.rsqrt(d[0] + d[1] + 1.0)
    dinv_ref[...] = dinv
    h = jnp.dot(x_ref[...], w_ref[...], preferred_element_type=jnp.float32)
    h_ref[...] = h * dinv


def _tc_mid_body(acc_ref, hp_ref, dinv_ref, b_ref, w_ref, out_ref):
    a = acc_ref[...]
    dinv = dinv_ref[...]
    agg = (a[0] + a[1] + hp_ref[...]) * dinv + b_ref[...]
    h2 = jnp.maximum(agg, 0.0)
    out_ref[...] = jnp.dot(h2, w_ref[...], preferred_element_type=jnp.float32) * dinv


def _tc_final_body(acc_ref, hp_ref, dinv_ref, b_ref, out_ref):
    a = acc_ref[...]
    out_ref[...] = (a[0] + a[1] + hp_ref[...]) * dinv_ref[...] + b_ref[...]


def kernel(x, edge_index, W1, b1, W2, b2):
    n = x.shape[0]
    e = edge_index.shape[1]
    n_pad = _pad_node_count(n)
    cpt = ((e + NW * CHUNK - 1) // (NW * CHUNK) + STAGE - 1) // STAGE * STAGE
    e_pad = NW * CHUNK * cpt
    ep_w = e_pad // NW

    src = edge_index[0].astype(jnp.int32)
    dst = edge_index[1].astype(jnp.int32)
    pad_idx = jnp.full((e_pad - e,), n, jnp.int32)
    src_p = jnp.concatenate([src, pad_idx])
    dst_p = jnp.concatenate([dst, pad_idx])
    x_p = jnp.pad(x, ((0, n_pad - n), (0, 0)))
    zeros = jnp.zeros((n_pad, CH), jnp.float32)

    deg_kernel = _make_deg_kernel(n_pad, ep_w)
    edge_kernel = _make_edge_kernel(n_pad, cpt)

    deg_p = deg_kernel(dst_p).reshape(NC, n_pad, 1)

    dinv, h1p = pl.pallas_call(
        _tc_prescale_body,
        out_shape=[
            jax.ShapeDtypeStruct((n_pad, 1), jnp.float32),
            jax.ShapeDtypeStruct((n_pad, CH), jnp.float32),
        ],
    )(deg_p, x_p, W1)

    acc1 = edge_kernel(h1p, src_p, dst_p, zeros)

    h2p = pl.pallas_call(
        _tc_mid_body,
        out_shape=jax.ShapeDtypeStruct((n_pad, CH), jnp.float32),
    )(acc1, h1p, dinv, b1.reshape(1, CH), W2)

    acc2 = edge_kernel(h2p, src_p, dst_p, zeros)

    out = pl.pallas_call(
        _tc_final_body,
        out_shape=jax.ShapeDtypeStruct((n_pad, CH), jnp.float32),
    )(acc2, h2p, dinv, b2.reshape(1, CH))

    return out[:n]


# speedup vs baseline: 1.4584x; 1.3607x over previous
import functools

import jax
import jax.numpy as jnp
from jax import lax
from jax.experimental import pallas as pl
from jax.experimental.pallas import tpu as pltpu
from jax.experimental.pallas import tpu_sc as plsc

CH = 128
NC = 2
NS = 16
NW = NC * NS
CHUNK = 128
STAGE = 16

_mesh = plsc.VectorSubcoreMesh(
    core_axis_name="c", subcore_axis_name="s", num_cores=NC, num_subcores=NS)


def _pad_node_count(n):
    m = 16 * NS
    return ((n + 1 + m - 1) // m) * m


def _make_deg_kernel(n_pad, ep_w):
    n_pt = n_pad // NS

    @functools.partial(
        pl.kernel,
        out_type=jax.ShapeDtypeStruct((NC, n_pad), jnp.float32),
        mesh=_mesh,
        scratch_types=[
            pltpu.VMEM((CHUNK,), jnp.int32),
            pltpu.VMEM((CHUNK,), jnp.float32),
            pltpu.VMEM((n_pt,), jnp.float32),
            pltpu.VMEM_SHARED((n_pad,), jnp.float32),
        ],
    )
    def deg_kernel(dst_hbm, out_hbm, didx_v, ones_v, zbuf_v, acc_sh):
        cid = lax.axis_index("c")
        sid = lax.axis_index("s")
        wid = cid * NS + sid
        for i in range(CHUNK // 16):
            ones_v[pl.ds(i * 16, 16)] = jnp.ones((16,), jnp.float32)
        for i in range(n_pt // 16):
            zbuf_v[pl.ds(i * 16, 16)] = jnp.zeros((16,), jnp.float32)
        pltpu.sync_copy(zbuf_v, acc_sh.at[pl.ds(sid * n_pt, n_pt)])
        plsc.subcore_barrier()

        def body(i, carry):
            base = wid * ep_w + i * CHUNK
            pltpu.sync_copy(dst_hbm.at[pl.ds(base, CHUNK)], didx_v)
            pltpu.sync_copy(ones_v, acc_sh.at[didx_v], add=True)
            return carry

        lax.fori_loop(0, ep_w // CHUNK, body, 0)
        plsc.subcore_barrier()
        pltpu.sync_copy(acc_sh.at[pl.ds(sid * n_pt, n_pt)],
                        out_hbm.at[cid, pl.ds(sid * n_pt, n_pt)])

    return deg_kernel


def _make_edge_kernel(n_pad, cpt0, cpt1):
    n_pt = n_pad // NS

    @functools.partial(
        pl.kernel,
        out_type=jax.ShapeDtypeStruct((NC, n_pad, CH), jnp.float32),
        mesh=_mesh,
        scratch_types=[
            pltpu.VMEM((CHUNK,), jnp.int32),
            pltpu.VMEM((CHUNK,), jnp.int32),
            pltpu.VMEM((CHUNK, CH), jnp.float32),
            pltpu.VMEM_SHARED((n_pad, CH), jnp.float32),
            pltpu.SemaphoreType.DMA,
        ],
    )
    def edge_kernel(tbl_hbm, src_hbm, dst_hbm, zeros_hbm, out_hbm,
                    sidx_v, didx_v, rows_v, acc_sh, sem):
        cid = lax.axis_index("c")
        sid = lax.axis_index("s")
        base0 = jnp.where(cid == 0, sid * cpt0, NS * cpt0 + sid * cpt1) * CHUNK
        trips = jnp.where(cid == 0, cpt0, cpt1)
        pltpu.sync_copy(zeros_hbm.at[pl.ds(sid * n_pt, n_pt), :],
                        acc_sh.at[pl.ds(sid * n_pt, n_pt), :])
        plsc.subcore_barrier()

        def body(i, carry):
            base = base0 + i * CHUNK
            pltpu.sync_copy(src_hbm.at[pl.ds(base, CHUNK)], sidx_v)
            pltpu.sync_copy(dst_hbm.at[pl.ds(base, CHUNK)], didx_v)
            pltpu.async_copy(tbl_hbm.at[sidx_v], rows_v, sem).wait()
            pltpu.sync_copy(rows_v, acc_sh.at[didx_v], add=True)
            return carry

        lax.fori_loop(0, trips, body, 0)
        plsc.subcore_barrier()
        pltpu.sync_copy(acc_sh.at[pl.ds(sid * n_pt, n_pt), :],
                        out_hbm.at[cid, pl.ds(sid * n_pt, n_pt), :])

    return edge_kernel


def _tc_prescale_body(deg_ref, x_ref, w_ref, dinv_ref, h_ref):
    d = deg_ref[...]
    dinv = lax.rsqrt(d[0] + d[1] + 1.0)
    dinv_ref[...] = dinv
    h = jnp.dot(x_ref[...], w_ref[...], preferred_element_type=jnp.float32)
    h_ref[...] = h * dinv


def _tc_mid_body(acc_ref, hp_ref, dinv_ref, b_ref, w_ref, out_ref):
    a = acc_ref[...]
    dinv = dinv_ref[...]
    agg = (a[0] + a[1] + hp_ref[...]) * dinv + b_ref[...]
    h2 = jnp.maximum(agg, 0.0)
    out_ref[...] = jnp.dot(h2, w_ref[...], preferred_element_type=jnp.float32) * dinv


def _tc_final_body(acc_ref, hp_ref, dinv_ref, b_ref, out_ref):
    a = acc_ref[...]
    out_ref[...] = (a[0] + a[1] + hp_ref[...]) * dinv_ref[...] + b_ref[...]


def kernel(x, edge_index, W1, b1, W2, b2):
    n = x.shape[0]
    e = edge_index.shape[1]
    n_pad = _pad_node_count(n)
    s = (e + NS * CHUNK - 1) // (NS * CHUNK)
    s += s % 2
    cpt1 = max(2, round(s * 256.0 / (256.0 + 362.0)))
    cpt0 = s - cpt1
    e_pad = NS * CHUNK * s
    ep_w = e_pad // NW

    src = edge_index[0].astype(jnp.int32)
    dst = edge_index[1].astype(jnp.int32)
    pad_idx = jnp.full((e_pad - e,), n, jnp.int32)
    src_p = jnp.concatenate([src, pad_idx])
    dst_p = jnp.concatenate([dst, pad_idx])
    x_p = jnp.pad(x, ((0, n_pad - n), (0, 0)))
    zeros = jnp.zeros((n_pad, CH), jnp.float32)

    deg_kernel = _make_deg_kernel(n_pad, ep_w)
    edge_kernel = _make_edge_kernel(n_pad, cpt0, cpt1)

    deg_p = deg_kernel(dst_p).reshape(NC, n_pad, 1)

    dinv, h1p = pl.pallas_call(
        _tc_prescale_body,
        out_shape=[
            jax.ShapeDtypeStruct((n_pad, 1), jnp.float32),
            jax.ShapeDtypeStruct((n_pad, CH), jnp.float32),
        ],
    )(deg_p, x_p, W1)

    acc1 = edge_kernel(h1p, src_p, dst_p, zeros)

    h2p = pl.pallas_call(
        _tc_mid_body,
        out_shape=jax.ShapeDtypeStruct((n_pad, CH), jnp.float32),
    )(acc1, h1p, dinv, b1.reshape(1, CH), W2)

    acc2 = edge_kernel(h2p, src_p, dst_p, zeros)

    out = pl.pallas_call(
        _tc_final_body,
        out_shape=jax.ShapeDtypeStruct((n_pad, CH), jnp.float32),
    )(acc2, h2p, dinv, b2.reshape(1, CH))

    return out[:n]


# speedup vs baseline: 7.4685x; 5.1209x over previous
import functools

import jax
import jax.numpy as jnp
from jax import lax
from jax.experimental import pallas as pl
from jax.experimental.pallas import tpu as pltpu
from jax.experimental.pallas import tpu_sc as plsc

CH = 128
NC = 2
NS = 16
NW = NC * NS
CHUNK = 128
STAGE = 16

_mesh = plsc.VectorSubcoreMesh(
    core_axis_name="c", subcore_axis_name="s", num_cores=NC, num_subcores=NS)


def _pad_node_count(n):
    m = 16 * NS
    return ((n + 1 + m - 1) // m) * m


def _make_deg_kernel(n_pad, ep_w):
    n_pt = n_pad // NS

    @functools.partial(
        pl.kernel,
        out_type=jax.ShapeDtypeStruct((NC, n_pad), jnp.float32),
        mesh=_mesh,
        scratch_types=[
            pltpu.VMEM((CHUNK,), jnp.int32),
            pltpu.VMEM((CHUNK,), jnp.float32),
            pltpu.VMEM((n_pt,), jnp.float32),
            pltpu.VMEM_SHARED((n_pad,), jnp.float32),
        ],
    )
    def deg_kernel(dst_hbm, out_hbm, didx_v, ones_v, zbuf_v, acc_sh):
        cid = lax.axis_index("c")
        sid = lax.axis_index("s")
        wid = cid * NS + sid
        for i in range(CHUNK // 16):
            ones_v[pl.ds(i * 16, 16)] = jnp.ones((16,), jnp.float32)
        for i in range(n_pt // 16):
            zbuf_v[pl.ds(i * 16, 16)] = jnp.zeros((16,), jnp.float32)
        pltpu.sync_copy(zbuf_v, acc_sh.at[pl.ds(sid * n_pt, n_pt)])
        plsc.subcore_barrier()

        def body(i, carry):
            base = wid * ep_w + i * CHUNK
            pltpu.sync_copy(dst_hbm.at[pl.ds(base, CHUNK)], didx_v)
            pltpu.sync_copy(ones_v, acc_sh.at[didx_v], add=True)
            return carry

        lax.fori_loop(0, ep_w // CHUNK, body, 0)
        plsc.subcore_barrier()
        pltpu.sync_copy(acc_sh.at[pl.ds(sid * n_pt, n_pt)],
                        out_hbm.at[cid, pl.ds(sid * n_pt, n_pt)])

    return deg_kernel


def _make_edge_kernel(n_pad, cpt0, cpt1):
    n_pt = n_pad // NS

    @functools.partial(
        pl.kernel,
        out_type=jax.ShapeDtypeStruct((NC, n_pad, CH), jnp.float32),
        mesh=_mesh,
        scratch_types=[
            pltpu.VMEM((2, CHUNK), jnp.int32),
            pltpu.VMEM((2, CHUNK), jnp.int32),
            pltpu.VMEM((2, CHUNK, CH), jnp.float32),
            pltpu.VMEM_SHARED((n_pad, CH), jnp.float32),
        ],
    )
    def edge_kernel(tbl_hbm, src_hbm, dst_hbm, zeros_hbm, out_hbm,
                    sidx_v, didx_v, rows_v, acc_sh):
        cid = lax.axis_index("c")
        sid = lax.axis_index("s")
        base0 = jnp.where(cid == 0, sid * cpt0, NS * cpt0 + sid * cpt1) * CHUNK
        trips = jnp.where(cid == 0, cpt0, cpt1)
        pltpu.sync_copy(zeros_hbm.at[pl.ds(sid * n_pt, n_pt), :],
                        acc_sh.at[pl.ds(sid * n_pt, n_pt), :])
        plsc.subcore_barrier()

        def body(k, carry):
            @functools.partial(plsc.parallel_loop, 0, 2)
            def inner(g):
                base = base0 + (2 * k + g) * CHUNK
                pltpu.sync_copy(src_hbm.at[pl.ds(base, CHUNK)], sidx_v.at[g])
                pltpu.sync_copy(dst_hbm.at[pl.ds(base, CHUNK)], didx_v.at[g])
                pltpu.sync_copy(tbl_hbm.at[sidx_v.at[g]], rows_v.at[g])
                pltpu.sync_copy(rows_v.at[g], acc_sh.at[didx_v.at[g]], add=True)

            return carry

        lax.fori_loop(0, trips // 2, body, 0)
        plsc.subcore_barrier()
        pltpu.sync_copy(acc_sh.at[pl.ds(sid * n_pt, n_pt), :],
                        out_hbm.at[cid, pl.ds(sid * n_pt, n_pt), :])

    return edge_kernel


def _tc_prescale_body(deg_ref, x_ref, w_ref, dinv_ref, h_ref):
    d = deg_ref[...]
    dinv = lax.rsqrt(d[0] + d[1] + 1.0)
    dinv_ref[...] = dinv
    h = jnp.dot(x_ref[...], w_ref[...], preferred_element_type=jnp.float32)
    h_ref[...] = h * dinv


def _tc_mid_body(acc_ref, hp_ref, dinv_ref, b_ref, w_ref, out_ref):
    a = acc_ref[...]
    dinv = dinv_ref[...]
    agg = (a[0] + a[1] + hp_ref[...]) * dinv + b_ref[...]
    h2 = jnp.maximum(agg, 0.0)
    out_ref[...] = jnp.dot(h2, w_ref[...], preferred_element_type=jnp.float32) * dinv


def _tc_final_body(acc_ref, hp_ref, dinv_ref, b_ref, out_ref):
    a = acc_ref[...]
    out_ref[...] = (a[0] + a[1] + hp_ref[...]) * dinv_ref[...] + b_ref[...]


def kernel(x, edge_index, W1, b1, W2, b2):
    n = x.shape[0]
    e = edge_index.shape[1]
    n_pad = _pad_node_count(n)
    s = (e + NS * CHUNK - 1) // (NS * CHUNK)
    s += s % 2
    cpt1 = max(2, round(s * 256.0 / (256.0 + 362.0)))
    cpt1 += cpt1 % 2
    cpt0 = s - cpt1
    e_pad = NS * CHUNK * s
    ep_w = e_pad // NW

    src = edge_index[0].astype(jnp.int32)
    dst = edge_index[1].astype(jnp.int32)
    pad_idx = jnp.full((e_pad - e,), n, jnp.int32)
    src_p = jnp.concatenate([src, pad_idx])
    dst_p = jnp.concatenate([dst, pad_idx])
    x_p = jnp.pad(x, ((0, n_pad - n), (0, 0)))
    zeros = jnp.zeros((n_pad, CH), jnp.float32)

    deg_kernel = _make_deg_kernel(n_pad, ep_w)
    edge_kernel = _make_edge_kernel(n_pad, cpt0, cpt1)

    deg_p = deg_kernel(dst_p).reshape(NC, n_pad, 1)

    dinv, h1p = pl.pallas_call(
        _tc_prescale_body,
        out_shape=[
            jax.ShapeDtypeStruct((n_pad, 1), jnp.float32),
            jax.ShapeDtypeStruct((n_pad, CH), jnp.float32),
        ],
    )(deg_p, x_p, W1)

    acc1 = edge_kernel(h1p, src_p, dst_p, zeros)

    h2p = pl.pallas_call(
        _tc_mid_body,
        out_shape=jax.ShapeDtypeStruct((n_pad, CH), jnp.float32),
    )(acc1, h1p, dinv, b1.reshape(1, CH), W2)

    acc2 = edge_kernel(h2p, src_p, dst_p, zeros)

    out = pl.pallas_call(
        _tc_final_body,
        out_shape=jax.ShapeDtypeStruct((n_pad, CH), jnp.float32),
    )(acc2, h2p, dinv, b2.reshape(1, CH))

    return out[:n]
